# baseline (speedup 1.0000x reference)
import functools

import jax
import jax.numpy as jnp
from jax import lax
from jax.experimental import pallas as pl
from jax.experimental.pallas import tpu as pltpu
from jax.experimental.pallas import tpu_sc as plsc

B = 16384
F_SPARSE = 26
VOCAB = 100000
F_DENSE = 13

NUM_CORES = 2
NUM_SUBCORES = 16
NW = NUM_CORES * NUM_SUBCORES
BW = B // NW
CHUNKS = BW // 16
IDX_ROWS = F_SPARSE * BW // 128


@functools.partial(
    pl.kernel,
    out_type=jax.ShapeDtypeStruct((B,), jnp.float32),
    mesh=plsc.VectorSubcoreMesh(core_axis_name="c", subcore_axis_name="s"),
    scratch_types=[
        pltpu.VMEM((IDX_ROWS, 128), jnp.int32),
        pltpu.VMEM((IDX_ROWS, 128), jnp.float32),
        pltpu.VMEM((F_DENSE, BW), jnp.float32),
        pltpu.VMEM((F_DENSE, 16), jnp.float32),
        pltpu.VMEM((BW,), jnp.float32),
        pltpu.SemaphoreType.DMA,
    ],
)
def _linear_logit_sc(comb_hbm, xd_hbm, wsp_hbm, table_hbm, out_hbm,
                     idx_v, g_v, xd_v, w_v, out_v, sem):
    wid = lax.axis_index("s") * NUM_CORES + lax.axis_index("c")
    pltpu.sync_copy(comb_hbm.at[wid], idx_v)
    pltpu.sync_copy(xd_hbm.at[wid], xd_v)
    pltpu.sync_copy(wsp_hbm, w_v)
    copies = [
        pltpu.async_copy(table_hbm.at[idx_v.at[j]], g_v.at[j], sem)
        for j in range(IDX_ROWS)
    ]
    for cp in copies:
        cp.wait()
    wvecs = [w_v[d] for d in range(F_DENSE)]
    for c in range(CHUNKS):
        cb, co = c // 8, (c % 8) * 16
        acc = g_v[cb, pl.ds(co, 16)]
        for f in range(1, F_SPARSE):
            acc = acc + g_v[f * (BW // 128) + cb, pl.ds(co, 16)]
        for d in range(F_DENSE):
            acc = acc + xd_v[d, pl.ds(c * 16, 16)] * wvecs[d]
        out_v[pl.ds(c * 16, 16)] = acc
    pltpu.sync_copy(out_v, out_hbm.at[pl.ds(wid * BW, BW)])


def kernel(X_sparse, X_dense, tables, W):
    offs = (jnp.arange(F_SPARSE, dtype=jnp.int32) * VOCAB)[:, None]
    comb = X_sparse.astype(jnp.int32).T + offs
    comb = (comb.reshape(F_SPARSE, NW, BW)
                .transpose(1, 0, 2)
                .reshape(NW, IDX_ROWS, 128))
    xd = X_dense.T.reshape(F_DENSE, NW, BW).transpose(1, 0, 2)
    wsp = jnp.broadcast_to(W, (F_DENSE, 16)).astype(jnp.float32)
    table_flat = tables.reshape(F_SPARSE * VOCAB)
    out = _linear_logit_sc(comb, xd, wsp, table_flat)
    return out.reshape(B, 1)
